# baseline (speedup 1.0000x reference)
import jax
import jax.numpy as jnp
from jax.experimental import pallas as pl
from jax.experimental.pallas import tpu as pltpu


def _moe_dense_body(x_ref, lg_ref, w_ref, o_ref, wgt_ref):
    e = pl.program_id(0)

    @pl.when(e == 0)
    def _compute_gate_weights():
        logits = lg_ref[...]
        iota = jax.lax.broadcasted_iota(jnp.int32, logits.shape, 1)
        i1 = jnp.argmax(logits, axis=1, keepdims=True)
        m1 = jnp.max(logits, axis=1, keepdims=True)
        masked = jnp.where(iota == i1, -jnp.inf, logits)
        i2 = jnp.argmax(masked, axis=1, keepdims=True)
        m2 = jnp.max(masked, axis=1, keepdims=True)
        ex = jnp.exp(m2 - m1)
        s = 1.0 + ex
        wgt_ref[...] = (jnp.where(iota == i1, 1.0 / s, 0.0)
                        + jnp.where(iota == i2, ex / s, 0.0))

    wbf = w_ref[0].astype(jnp.bfloat16)
    wgt = wgt_ref[...]
    lane = jax.lax.broadcasted_iota(jnp.int32, wgt.shape, 1)
    wcol = jnp.sum(jnp.where(lane == e, wgt, 0.0), axis=1, keepdims=True)

    n_tokens = x_ref.shape[0]
    chunk = min(1024, n_tokens)
    for m in range(0, n_tokens, chunk):
        sl = slice(m, m + chunk)
        y = jax.lax.dot_general(
            x_ref[sl, :], wbf, (((1,), (1,)), ((), ())),
            preferred_element_type=jnp.float32)
        contrib = wcol[sl, :] * y

        @pl.when(e == 0)
        def _init(sl=sl, contrib=contrib):
            o_ref[sl, :] = contrib

        @pl.when(e != 0)
        def _accumulate(sl=sl, contrib=contrib):
            o_ref[sl, :] = o_ref[sl, :] + contrib


def kernel(inputs, gate_w, expert_w):
    B, T, C = inputs.shape
    E = gate_w.shape[0]
    N = B * T
    x = inputs.reshape(N, C)

    gate_logits = x @ gate_w.T
    x_bf = x.astype(jnp.bfloat16)

    out = pl.pallas_call(
        _moe_dense_body,
        out_shape=jax.ShapeDtypeStruct((N, C), jnp.float32),
        grid=(E,),
        in_specs=[
            pl.BlockSpec((N, C), lambda e: (0, 0)),
            pl.BlockSpec((N, E), lambda e: (0, 0)),
            pl.BlockSpec((1, C, C), lambda e: (e, 0, 0)),
        ],
        out_specs=pl.BlockSpec((N, C), lambda e: (0, 0)),
        scratch_shapes=[
            pltpu.VMEM((N, E), jnp.float32),
        ],
        compiler_params=pltpu.CompilerParams(
            dimension_semantics=("arbitrary",),
            vmem_limit_bytes=63 * 1024 * 1024,
        ),
    )(x_bf, gate_logits, expert_w)

    return out.astype(inputs.dtype).reshape(B, T, C)


# speedup vs baseline: 1.2217x; 1.2217x over previous
import jax
import jax.numpy as jnp
from jax.experimental import pallas as pl
from jax.experimental.pallas import tpu as pltpu

_TM = 1024


def _moe_dense_body(x_ref, lg_ref, w_hbm, o_ref, wbf_ref, stg_ref, sems):
    t = pl.program_id(0)
    num_experts = w_hbm.shape[0]

    logits = lg_ref[...]
    iota = jax.lax.broadcasted_iota(jnp.int32, logits.shape, 1)

    i1 = jnp.argmax(logits, axis=1, keepdims=True)
    m1 = jnp.max(logits, axis=1, keepdims=True)
    masked = jnp.where(iota == i1, -jnp.inf, logits)
    i2 = jnp.argmax(masked, axis=1, keepdims=True)
    m2 = jnp.max(masked, axis=1, keepdims=True)
    ex = jnp.exp(m2 - m1)
    s = 1.0 + ex
    wgt = (jnp.where(iota == i1, 1.0 / s, 0.0)
           + jnp.where(iota == i2, ex / s, 0.0))

    x = x_ref[...]

    def dot_e(e):
        y = jax.lax.dot_general(
            x, wbf_ref[e], (((1,), (1,)), ((), ())),
            preferred_element_type=jnp.float32)
        return wgt[:, e][:, None] * y

    @pl.when(t == 0)
    def _first_step():
        copies = [
            pltpu.make_async_copy(w_hbm.at[e], stg_ref.at[e % 2], sems.at[e % 2])
            for e in range(num_experts)
        ]
        copies[0].start()
        acc = None
        for e in range(num_experts):
            if e + 1 < num_experts:
                copies[e + 1].start()
            copies[e].wait()
            wbf_ref[e] = stg_ref[e % 2].astype(jnp.bfloat16)
            term = dot_e(e)
            acc = term if acc is None else acc + term
        o_ref[...] = acc

    @pl.when(t != 0)
    def _steady_step():
        acc = None
        for e in range(num_experts):
            term = dot_e(e)
            acc = term if acc is None else acc + term
        o_ref[...] = acc


def kernel(inputs, gate_w, expert_w):
    B, T, C = inputs.shape
    E = gate_w.shape[0]
    N = B * T
    x = inputs.reshape(N, C)

    gate_logits = x @ gate_w.T
    x_bf = x.astype(jnp.bfloat16)

    tm = _TM if N % _TM == 0 else N
    out = pl.pallas_call(
        _moe_dense_body,
        out_shape=jax.ShapeDtypeStruct((N, C), jnp.float32),
        grid=(N // tm,),
        in_specs=[
            pl.BlockSpec((tm, C), lambda t: (t, 0)),
            pl.BlockSpec((tm, E), lambda t: (t, 0)),
            pl.BlockSpec(memory_space=pl.ANY),
        ],
        out_specs=pl.BlockSpec((tm, C), lambda t: (t, 0)),
        scratch_shapes=[
            pltpu.VMEM((E, C, C), jnp.bfloat16),
            pltpu.VMEM((2, C, C), jnp.float32),
            pltpu.SemaphoreType.DMA((2,)),
        ],
        compiler_params=pltpu.CompilerParams(
            dimension_semantics=("arbitrary",),
            vmem_limit_bytes=63 * 1024 * 1024,
        ),
    )(x_bf, gate_logits, expert_w)

    return out.astype(inputs.dtype).reshape(B, T, C)


# speedup vs baseline: 1.2846x; 1.0515x over previous
import jax
import jax.numpy as jnp
from jax.experimental import pallas as pl
from jax.experimental.pallas import tpu as pltpu

_TM = 1024


def _moe_dense_body(x_ref, lg_ref, w_hbm, o_ref, wbf_ref, stg_ref, sems):
    t = pl.program_id(0)
    num_experts = w_hbm.shape[0]

    logits = lg_ref[...]
    iota = jax.lax.broadcasted_iota(jnp.int32, logits.shape, 1)

    i1 = jnp.argmax(logits, axis=1, keepdims=True)
    m1 = jnp.max(logits, axis=1, keepdims=True)
    masked = jnp.where(iota == i1, -jnp.inf, logits)
    i2 = jnp.argmax(masked, axis=1, keepdims=True)
    m2 = jnp.max(masked, axis=1, keepdims=True)
    ex = jnp.exp(m2 - m1)
    s = 1.0 + ex
    wgt = (jnp.where(iota == i1, 1.0 / s, 0.0)
           + jnp.where(iota == i2, ex / s, 0.0))

    x = x_ref[...].astype(jnp.bfloat16)

    def dot_e(e):
        y = jax.lax.dot_general(
            x, wbf_ref[e], (((1,), (1,)), ((), ())),
            preferred_element_type=jnp.float32)
        return wgt[:, e][:, None] * y

    @pl.when(t == 0)
    def _first_step():
        copies = [
            pltpu.make_async_copy(w_hbm.at[e], stg_ref.at[e % 2], sems.at[e % 2])
            for e in range(num_experts)
        ]
        copies[0].start()
        acc = None
        for e in range(num_experts):
            if e + 1 < num_experts:
                copies[e + 1].start()
            copies[e].wait()
            wbf_ref[e] = stg_ref[e % 2].astype(jnp.bfloat16)
            term = dot_e(e)
            acc = term if acc is None else acc + term
        o_ref[...] = acc

    @pl.when(t != 0)
    def _steady_step():
        acc = None
        for e in range(num_experts):
            term = dot_e(e)
            acc = term if acc is None else acc + term
        o_ref[...] = acc


def kernel(inputs, gate_w, expert_w):
    B, T, C = inputs.shape
    E = gate_w.shape[0]
    N = B * T
    x = inputs.reshape(N, C)

    gate_logits = x @ gate_w.T

    tm = _TM if N % _TM == 0 else N
    out = pl.pallas_call(
        _moe_dense_body,
        out_shape=jax.ShapeDtypeStruct((N, C), jnp.float32),
        grid=(N // tm,),
        in_specs=[
            pl.BlockSpec((tm, C), lambda t: (t, 0)),
            pl.BlockSpec((tm, E), lambda t: (t, 0)),
            pl.BlockSpec(memory_space=pl.ANY),
        ],
        out_specs=pl.BlockSpec((tm, C), lambda t: (t, 0)),
        scratch_shapes=[
            pltpu.VMEM((E, C, C), jnp.bfloat16),
            pltpu.VMEM((2, C, C), jnp.float32),
            pltpu.SemaphoreType.DMA((2,)),
        ],
        compiler_params=pltpu.CompilerParams(
            dimension_semantics=("arbitrary",),
            vmem_limit_bytes=63 * 1024 * 1024,
        ),
    )(x, gate_logits, expert_w)

    return out.astype(inputs.dtype).reshape(B, T, C)
